# baseline (speedup 1.0000x reference)
import functools

import jax
import jax.numpy as jnp
from jax import lax
from jax.experimental import pallas as pl
from jax.experimental.pallas import tpu as pltpu
from jax.experimental.pallas import tpu_sc as plsc

PAIR_DIM = 128
EMBED_DIM = 64
LANES = 16
NUM_CORES = 2
NUM_SUBCORES = 16
NUM_WORKERS = NUM_CORES * NUM_SUBCORES
CHUNK = 160
N_CHUNKS = 40

TBLOCK = 2048


def _transpose_block(in_ref, out_ref):
    x = in_ref[...]
    y = x.T.reshape(TBLOCK, 2, EMBED_DIM)
    out_ref[...] = jnp.concatenate([y[:, 0, :], y[:, 1, :]], axis=1)


def _relayout_table(wt):
    v = wt.shape[1]
    grid = (v // 2 + TBLOCK - 1) // TBLOCK
    return pl.pallas_call(
        _transpose_block,
        grid=(grid,),
        in_specs=[pl.BlockSpec((EMBED_DIM, 2 * TBLOCK), lambda n: (0, n))],
        out_specs=pl.BlockSpec((TBLOCK, PAIR_DIM), lambda n: (n, 0)),
        out_shape=jax.ShapeDtypeStruct((v // 2, PAIR_DIM), jnp.float32),
    )(wt)


def _make_gather(total_rows: int):
    rows_per_w = total_rows // NUM_WORKERS
    assert rows_per_w == CHUNK * N_CHUNKS
    mesh = plsc.VectorSubcoreMesh(core_axis_name="c", subcore_axis_name="s")

    @functools.partial(
        pl.kernel,
        mesh=mesh,
        out_type=jax.ShapeDtypeStruct((total_rows, EMBED_DIM), jnp.float32),
        scratch_types=[
            pltpu.VMEM((rows_per_w,), jnp.int32),
            pltpu.VMEM((rows_per_w,), jnp.int32),
            pltpu.VMEM((CHUNK, PAIR_DIM), jnp.float32),
            pltpu.VMEM((CHUNK, PAIR_DIM), jnp.float32),
            pltpu.VMEM((CHUNK, PAIR_DIM), jnp.float32),
            pltpu.VMEM((CHUNK, EMBED_DIM), jnp.float32),
            pltpu.VMEM((CHUNK, EMBED_DIM), jnp.float32),
            pltpu.SemaphoreType.DMA,
            pltpu.SemaphoreType.DMA,
            pltpu.SemaphoreType.DMA,
            pltpu.SemaphoreType.DMA,
            pltpu.SemaphoreType.DMA,
        ],
    )
    def gather(
        table_hbm, idx_hbm, out_hbm,
        idx_v, pair_v, rows0, rows1, rows2, comp0, comp1, g0, g1, g2, o0, o1,
    ):
        wid = lax.axis_index("s") * NUM_CORES + lax.axis_index("c")
        base = wid * rows_per_w
        pltpu.sync_copy(idx_hbm.at[pl.ds(base, rows_per_w)], idx_v)

        def halve(j, carry):
            idx16 = idx_v[pl.ds(j * LANES, LANES)]
            pair_v[pl.ds(j * LANES, LANES)] = lax.shift_right_logical(idx16, 1)
            return carry

        lax.fori_loop(0, rows_per_w // LANES, halve, 0)

        rows = (rows0, rows1, rows2)
        comp = (comp0, comp1)
        gsem = (g0, g1, g2)
        osem = (o0, o1)

        def start_gather(i):
            return pltpu.async_copy(
                table_hbm.at[pair_v.at[pl.ds(i * CHUNK, CHUNK)]],
                rows[i % 3],
                gsem[i % 3],
            )

        def start_out(i):
            return pltpu.async_copy(
                comp[i % 2], out_hbm.at[pl.ds(base + i * CHUNK, CHUNK)], osem[i % 2]
            )

        def compact(i):
            src = rows[i % 3]
            dst = comp[i % 2]

            def group(gi, carry):
                r0 = gi * LANES
                idx16 = idx_v[pl.ds(i * CHUNK + r0, LANES)]
                for j in range(LANES):
                    off = (idx16[j] & 1) * EMBED_DIM
                    for k in range(EMBED_DIM // LANES):
                        dst[r0 + j, pl.ds(k * LANES, LANES)] = src[
                            r0 + j, pl.ds(off + k * LANES, LANES)
                        ]
                return carry

            lax.fori_loop(0, CHUNK // LANES, group, 0)

        g = [None] * N_CHUNKS
        o = [None] * N_CHUNKS
        g[0] = start_gather(0)
        g[1] = start_gather(1)
        for i in range(N_CHUNKS):
            g[i].wait()
            if i + 2 < N_CHUNKS:
                g[i + 2] = start_gather(i + 2)
            if i >= 2:
                o[i - 2].wait()
            compact(i)
            o[i] = start_out(i)
        o[N_CHUNKS - 2].wait()
        o[N_CHUNKS - 1].wait()

    return gather


def kernel(IX, weight):
    b, t = IX.shape
    total = b * t
    table = _relayout_table(weight.T)
    idx = IX.reshape(-1).astype(jnp.int32)
    out = _make_gather(total)(table, idx)
    return out.reshape(b, t, EMBED_DIM)


# speedup vs baseline: 1.1791x; 1.1791x over previous
import functools

import jax
import jax.numpy as jnp
from jax import lax
from jax.experimental import pallas as pl
from jax.experimental.pallas import tpu as pltpu
from jax.experimental.pallas import tpu_sc as plsc

PAD_DIM = 128
EMBED_DIM = 64
NUM_CORES = 2
NUM_SUBCORES = 16
NUM_WORKERS = NUM_CORES * NUM_SUBCORES
CHUNK = 400
N_CHUNKS = 16

TBLOCK = 4096


def _transpose_block(in_ref, eye_ref, out_ref):
    x = in_ref[...]
    out_ref[...] = lax.dot_general(
        x,
        eye_ref[...],
        (((0,), (0,)), ((), ())),
        preferred_element_type=jnp.float32,
    )


def _relayout_table(wt):
    v = wt.shape[1]
    grid = (v + TBLOCK - 1) // TBLOCK
    eye = jnp.eye(EMBED_DIM, PAD_DIM, dtype=jnp.float32)
    return pl.pallas_call(
        _transpose_block,
        grid=(grid,),
        in_specs=[
            pl.BlockSpec((EMBED_DIM, TBLOCK), lambda n: (0, n)),
            pl.BlockSpec((EMBED_DIM, PAD_DIM), lambda n: (0, 0)),
        ],
        out_specs=pl.BlockSpec((TBLOCK, PAD_DIM), lambda n: (n, 0)),
        out_shape=jax.ShapeDtypeStruct((v, PAD_DIM), jnp.float32),
    )(wt, eye)


def _make_gather(total_rows: int):
    rows_per_w = total_rows // NUM_WORKERS
    assert rows_per_w == CHUNK * N_CHUNKS
    mesh = plsc.VectorSubcoreMesh(core_axis_name="c", subcore_axis_name="s")

    @functools.partial(
        pl.kernel,
        mesh=mesh,
        out_type=jax.ShapeDtypeStruct((total_rows, PAD_DIM), jnp.float32),
        scratch_types=[
            pltpu.VMEM((rows_per_w,), jnp.int32),
            pltpu.VMEM((CHUNK, PAD_DIM), jnp.float32),
            pltpu.VMEM((CHUNK, PAD_DIM), jnp.float32),
            pltpu.SemaphoreType.DMA,
            pltpu.SemaphoreType.DMA,
            pltpu.SemaphoreType.DMA,
            pltpu.SemaphoreType.DMA,
        ],
    )
    def gather(table_hbm, idx_hbm, out_hbm, idx_v, rows0, rows1, g0, g1, o0, o1):
        wid = lax.axis_index("s") * NUM_CORES + lax.axis_index("c")
        base = wid * rows_per_w
        pltpu.sync_copy(idx_hbm.at[pl.ds(base, rows_per_w)], idx_v)

        rows = (rows0, rows1)
        gsem = (g0, g1)
        osem = (o0, o1)

        def start_gather(i):
            return pltpu.async_copy(
                table_hbm.at[idx_v.at[pl.ds(i * CHUNK, CHUNK)]],
                rows[i % 2],
                gsem[i % 2],
            )

        def start_out(i):
            return pltpu.async_copy(
                rows[i % 2], out_hbm.at[pl.ds(base + i * CHUNK, CHUNK)], osem[i % 2]
            )

        g = [None] * N_CHUNKS
        o = [None] * N_CHUNKS
        g[0] = start_gather(0)
        g[1] = start_gather(1)
        for i in range(N_CHUNKS):
            g[i].wait()
            o[i] = start_out(i)
            if i + 2 < N_CHUNKS:
                o[i].wait()
                g[i + 2] = start_gather(i + 2)
        o[N_CHUNKS - 2].wait()
        o[N_CHUNKS - 1].wait()

    return gather


def kernel(IX, weight):
    b, t = IX.shape
    total = b * t
    wp = _relayout_table(weight.T)
    idx = IX.reshape(-1).astype(jnp.int32)
    out = _make_gather(total)(wp, idx)
    return out[:, :EMBED_DIM].reshape(b, t, EMBED_DIM)


# speedup vs baseline: 1.4212x; 1.2053x over previous
import functools

import jax
import jax.numpy as jnp
from jax import lax
from jax.experimental import pallas as pl
from jax.experimental.pallas import tpu as pltpu
from jax.experimental.pallas import tpu_sc as plsc

PAD_DIM = 128
EMBED_DIM = 64
NUM_CORES = 2
NUM_SUBCORES = 16
NUM_WORKERS = NUM_CORES * NUM_SUBCORES
CHUNK = 200
N_CHUNKS = 32
NBUF = 4

TBLOCK = 16384


def _transpose_block(in_ref, out_ref):
    x = in_ref[...]
    y = x.T
    out_ref[...] = jnp.concatenate(
        [y, jnp.zeros((TBLOCK, PAD_DIM - EMBED_DIM), jnp.float32)], axis=1
    )


def _relayout_table(wt):
    v = wt.shape[1]
    grid = (v + TBLOCK - 1) // TBLOCK
    return pl.pallas_call(
        _transpose_block,
        grid=(grid,),
        in_specs=[pl.BlockSpec((EMBED_DIM, TBLOCK), lambda n: (0, n))],
        out_specs=pl.BlockSpec((TBLOCK, PAD_DIM), lambda n: (n, 0)),
        out_shape=jax.ShapeDtypeStruct((v, PAD_DIM), jnp.float32),
    )(wt)


def _make_gather(total_rows: int):
    rows_per_w = total_rows // NUM_WORKERS
    assert rows_per_w == CHUNK * N_CHUNKS
    mesh = plsc.VectorSubcoreMesh(core_axis_name="c", subcore_axis_name="s")

    @functools.partial(
        pl.kernel,
        mesh=mesh,
        out_type=jax.ShapeDtypeStruct((total_rows, PAD_DIM), jnp.float32),
        scratch_types=[
            pltpu.VMEM((rows_per_w,), jnp.int32),
            pltpu.VMEM((NBUF, CHUNK, PAD_DIM), jnp.float32),
            pltpu.SemaphoreType.DMA,
            pltpu.SemaphoreType.DMA,
            pltpu.SemaphoreType.DMA,
            pltpu.SemaphoreType.DMA,
            pltpu.SemaphoreType.DMA,
            pltpu.SemaphoreType.DMA,
            pltpu.SemaphoreType.DMA,
            pltpu.SemaphoreType.DMA,
        ],
    )
    def gather(table_hbm, idx_hbm, out_hbm, idx_v, rows_v, *sems):
        wid = lax.axis_index("s") * NUM_CORES + lax.axis_index("c")
        base = wid * rows_per_w
        pltpu.sync_copy(idx_hbm.at[pl.ds(base, rows_per_w)], idx_v)

        gsem = sems[:NBUF]
        osem = sems[NBUF:]

        def start_gather(i):
            return pltpu.async_copy(
                table_hbm.at[idx_v.at[pl.ds(i * CHUNK, CHUNK)]],
                rows_v.at[i % NBUF],
                gsem[i % NBUF],
            )

        def start_out(i):
            return pltpu.async_copy(
                rows_v.at[i % NBUF],
                out_hbm.at[pl.ds(base + i * CHUNK, CHUNK)],
                osem[i % NBUF],
            )

        g = [None] * N_CHUNKS
        o = [None] * N_CHUNKS
        for i in range(NBUF - 1):
            g[i] = start_gather(i)
        for i in range(N_CHUNKS):
            g[i].wait()
            o[i] = start_out(i)
            if i + NBUF - 1 < N_CHUNKS:
                if i >= 1:
                    o[i - 1].wait()
                g[i + NBUF - 1] = start_gather(i + NBUF - 1)
        for i in range(N_CHUNKS - NBUF, N_CHUNKS):
            o[i].wait()

    return gather


def kernel(IX, weight):
    b, t = IX.shape
    total = b * t
    wp = _relayout_table(weight.T)
    idx = IX.reshape(-1).astype(jnp.int32)
    out = _make_gather(total)(wp, idx)
    return out[:, :EMBED_DIM].reshape(b, t, EMBED_DIM)


# speedup vs baseline: 1.4404x; 1.0135x over previous
import functools

import jax
import jax.numpy as jnp
from jax import lax
from jax.experimental import pallas as pl
from jax.experimental.pallas import tpu as pltpu
from jax.experimental.pallas import tpu_sc as plsc

PAD_DIM = 128
EMBED_DIM = 64
NUM_CORES = 2
NUM_SUBCORES = 16
NUM_WORKERS = NUM_CORES * NUM_SUBCORES
CHUNK = 128
N_CHUNKS = 50
NBUF = 6

TBLOCK = 32768


def _transpose_block(in_ref, out_ref):
    x = in_ref[...]
    y = x.T
    out_ref[...] = jnp.concatenate(
        [y, jnp.zeros((TBLOCK, PAD_DIM - EMBED_DIM), jnp.float32)], axis=1
    )


def _relayout_table(wt):
    v = wt.shape[1]
    grid = (v + TBLOCK - 1) // TBLOCK
    return pl.pallas_call(
        _transpose_block,
        grid=(grid,),
        in_specs=[pl.BlockSpec((EMBED_DIM, TBLOCK), lambda n: (0, n))],
        out_specs=pl.BlockSpec((TBLOCK, PAD_DIM), lambda n: (n, 0)),
        out_shape=jax.ShapeDtypeStruct((v, PAD_DIM), jnp.float32),
    )(wt)


def _make_gather(total_rows: int):
    rows_per_w = total_rows // NUM_WORKERS
    assert rows_per_w == CHUNK * N_CHUNKS
    mesh = plsc.VectorSubcoreMesh(core_axis_name="c", subcore_axis_name="s")

    @functools.partial(
        pl.kernel,
        mesh=mesh,
        out_type=jax.ShapeDtypeStruct((total_rows, PAD_DIM), jnp.float32),
        scratch_types=[
            pltpu.VMEM((rows_per_w,), jnp.int32),
            pltpu.VMEM((NBUF, CHUNK, PAD_DIM), jnp.float32),
        ]
        + [pltpu.SemaphoreType.DMA] * (2 * NBUF),
    )
    def gather(table_hbm, idx_hbm, out_hbm, idx_v, rows_v, *sems):
        wid = lax.axis_index("s") * NUM_CORES + lax.axis_index("c")
        base = wid * rows_per_w
        pltpu.sync_copy(idx_hbm.at[pl.ds(base, rows_per_w)], idx_v)

        gsem = sems[:NBUF]
        osem = sems[NBUF:]

        def start_gather(i):
            return pltpu.async_copy(
                table_hbm.at[idx_v.at[pl.ds(i * CHUNK, CHUNK)]],
                rows_v.at[i % NBUF],
                gsem[i % NBUF],
            )

        def start_out(i):
            return pltpu.async_copy(
                rows_v.at[i % NBUF],
                out_hbm.at[pl.ds(base + i * CHUNK, CHUNK)],
                osem[i % NBUF],
            )

        g = [None] * N_CHUNKS
        o = [None] * N_CHUNKS
        for i in range(NBUF - 1):
            g[i] = start_gather(i)
        for i in range(N_CHUNKS):
            g[i].wait()
            o[i] = start_out(i)
            if i + NBUF - 1 < N_CHUNKS:
                if i >= 1:
                    o[i - 1].wait()
                g[i + NBUF - 1] = start_gather(i + NBUF - 1)
        for i in range(N_CHUNKS - NBUF, N_CHUNKS):
            o[i].wait()

    return gather


def kernel(IX, weight):
    b, t = IX.shape
    total = b * t
    wp = _relayout_table(weight.T)
    idx = IX.reshape(-1).astype(jnp.int32)
    out = _make_gather(total)(wp, idx)
    return out[:, :EMBED_DIM].reshape(b, t, EMBED_DIM)


# speedup vs baseline: 1.4425x; 1.0015x over previous
import functools

import jax
import jax.numpy as jnp
from jax import lax
from jax.experimental import pallas as pl
from jax.experimental.pallas import tpu as pltpu
from jax.experimental.pallas import tpu_sc as plsc

PAD_DIM = 128
EMBED_DIM = 64
NUM_CORES = 2
NUM_SUBCORES = 16
NUM_WORKERS = NUM_CORES * NUM_SUBCORES
CHUNK = 128
N_CHUNKS = 50
NBUF = 7

TBLOCK = 32768


def _transpose_block(in_ref, out_ref):
    x = in_ref[...]
    y = x.T
    out_ref[...] = jnp.concatenate(
        [y, jnp.zeros((TBLOCK, PAD_DIM - EMBED_DIM), jnp.float32)], axis=1
    )


def _relayout_table(wt):
    v = wt.shape[1]
    grid = (v + TBLOCK - 1) // TBLOCK
    return pl.pallas_call(
        _transpose_block,
        grid=(grid,),
        in_specs=[pl.BlockSpec((EMBED_DIM, TBLOCK), lambda n: (0, n))],
        out_specs=pl.BlockSpec((TBLOCK, PAD_DIM), lambda n: (n, 0)),
        out_shape=jax.ShapeDtypeStruct((v, PAD_DIM), jnp.float32),
    )(wt)


def _make_gather(total_rows: int):
    rows_per_w = total_rows // NUM_WORKERS
    assert rows_per_w == CHUNK * N_CHUNKS
    mesh = plsc.VectorSubcoreMesh(core_axis_name="c", subcore_axis_name="s")

    @functools.partial(
        pl.kernel,
        mesh=mesh,
        out_type=jax.ShapeDtypeStruct((total_rows, PAD_DIM), jnp.float32),
        scratch_types=[
            pltpu.VMEM((rows_per_w,), jnp.int32),
            pltpu.VMEM((NBUF, CHUNK, PAD_DIM), jnp.float32),
        ]
        + [pltpu.SemaphoreType.DMA] * (2 * NBUF),
    )
    def gather(table_hbm, idx_hbm, out_hbm, idx_v, rows_v, *sems):
        wid = lax.axis_index("s") * NUM_CORES + lax.axis_index("c")
        base = wid * rows_per_w
        pltpu.sync_copy(idx_hbm.at[pl.ds(base, rows_per_w)], idx_v)

        gsem = sems[:NBUF]
        osem = sems[NBUF:]

        def start_gather(i):
            return pltpu.async_copy(
                table_hbm.at[idx_v.at[pl.ds(i * CHUNK, CHUNK)]],
                rows_v.at[i % NBUF],
                gsem[i % NBUF],
            )

        def start_out(i):
            return pltpu.async_copy(
                rows_v.at[i % NBUF],
                out_hbm.at[pl.ds(base + i * CHUNK, CHUNK)],
                osem[i % NBUF],
            )

        g = [None] * N_CHUNKS
        o = [None] * N_CHUNKS
        for i in range(NBUF - 1):
            g[i] = start_gather(i)
        for i in range(N_CHUNKS):
            g[i].wait()
            o[i] = start_out(i)
            if i + NBUF - 1 < N_CHUNKS:
                if i >= 1:
                    o[i - 1].wait()
                g[i + NBUF - 1] = start_gather(i + NBUF - 1)
        for i in range(N_CHUNKS - NBUF, N_CHUNKS):
            o[i].wait()

    return gather


def kernel(IX, weight):
    b, t = IX.shape
    total = b * t
    wp = _relayout_table(weight.T)
    idx = IX.reshape(-1).astype(jnp.int32)
    out = _make_gather(total)(wp, idx)
    return out[:, :EMBED_DIM].reshape(b, t, EMBED_DIM)
